# baseline (speedup 1.0000x reference)
import functools

import jax
import jax.numpy as jnp
from jax import lax
from jax.experimental import pallas as pl
from jax.experimental.pallas import tpu as pltpu
from jax.experimental.pallas import tpu_sc as plsc

NC = 2
NS = 16
NW = NC * NS
EB = 128
D = 128
N = 10000
RPT = 640
NPAD = NS * RPT
R = 2000
NSLOT = 8
GSLOT = 2
LAG = 1


def _vsmesh():
    return plsc.VectorSubcoreMesh(core_axis_name="c", subcore_axis_name="s")


def _sc_degree(col2d):
    er = col2d.shape[0]
    nb = er // NW

    @functools.partial(
        pl.kernel,
        mesh=_vsmesh(),
        out_type=jax.ShapeDtypeStruct((NC * NPAD,), jnp.float32),
        scratch_types=[
            pltpu.VMEM((NSLOT, EB), jnp.int32),
            pltpu.VMEM((EB,), jnp.float32),
            pltpu.VMEM((RPT,), jnp.float32),
            pltpu.VMEM_SHARED((NPAD,), jnp.float32),
            pltpu.SemaphoreType.DMA((NSLOT,)),
        ],
    )
    def deg_kernel(col_hbm, out_hbm, colv, onesv, stg, deg_sh, csem):
        c = lax.axis_index("c")
        s = lax.axis_index("s")
        wid = s * NC + c
        for j in range(EB // 16):
            onesv[pl.ds(j * 16, 16)] = jnp.ones((16,), jnp.float32)
        for j in range(RPT // 16):
            stg[pl.ds(j * 16, 16)] = jnp.zeros((16,), jnp.float32)
        pltpu.sync_copy(stg, deg_sh.at[pl.ds(s * RPT, RPT)])
        plsc.subcore_barrier()

        def fire_idx(j, b):
            pltpu.async_copy(col_hbm.at[wid + j * NW], colv.at[b], csem.at[b])

        def wait_idx(j, b):
            pltpu.make_async_copy(
                col_hbm.at[wid + j * NW], colv.at[b], csem.at[b]
            ).wait()

        for b in range(NSLOT - 1):
            fire_idx(b, b)

        def body(i, carry):
            for b in range(NSLOT):
                j = i * NSLOT + b
                wait_idx(j, b)
                pltpu.sync_copy(onesv, deg_sh.at[colv.at[b]], add=True)

                @pl.when(j + NSLOT - 1 < nb)
                def _():
                    fire_idx(j + NSLOT - 1, (b + NSLOT - 1) % NSLOT)

            return carry

        lax.fori_loop(0, nb // NSLOT, body, 0)
        plsc.subcore_barrier()
        pltpu.sync_copy(
            deg_sh.at[pl.ds(s * RPT, RPT)],
            out_hbm.at[pl.ds(c * NPAD + s * RPT, RPT)],
        )

    return deg_kernel(col2d)


def _sc_spmm(q, row2d, col2d, zrows):
    er = row2d.shape[0]
    nb = er // NW

    @functools.partial(
        pl.kernel,
        mesh=_vsmesh(),
        out_type=jax.ShapeDtypeStruct((NC * NPAD, D), jnp.float32),
        scratch_types=[
            pltpu.VMEM((NSLOT, EB), jnp.int32),
            pltpu.VMEM((NSLOT, EB), jnp.int32),
            pltpu.VMEM((GSLOT, EB, D), jnp.float32),
            pltpu.VMEM_SHARED((NPAD, D), jnp.float32),
            pltpu.SemaphoreType.DMA((NSLOT,)),
            pltpu.SemaphoreType.DMA((NSLOT,)),
            pltpu.SemaphoreType.DMA((GSLOT,)),
            pltpu.SemaphoreType.DMA((GSLOT,)),
        ],
    )
    def spmm_kernel(
        q_hbm, row_hbm, col_hbm, z_hbm, out_hbm, rowv, colv, gbuf, acc_sh,
        rsem, csem, gsem, ssem,
    ):
        c = lax.axis_index("c")
        s = lax.axis_index("s")
        wid = s * NC + c

        pltpu.sync_copy(z_hbm, acc_sh.at[pl.ds(s * RPT, RPT)])
        plsc.subcore_barrier()

        def fire_idx(j, b):
            pltpu.async_copy(row_hbm.at[wid + j * NW], rowv.at[b], rsem.at[b])
            pltpu.async_copy(col_hbm.at[wid + j * NW], colv.at[b], csem.at[b])

        def wait_row(j, b):
            pltpu.make_async_copy(
                row_hbm.at[wid + j * NW], rowv.at[b], rsem.at[b]
            ).wait()

        def fire_gather(b, g):
            pltpu.async_copy(q_hbm.at[rowv.at[b]], gbuf.at[g], gsem.at[g])

        def fire_scatter(jj, b, g):
            pltpu.make_async_copy(
                q_hbm.at[rowv.at[b]], gbuf.at[g], gsem.at[g]
            ).wait()
            pltpu.make_async_copy(
                col_hbm.at[wid + jj * NW], colv.at[b], csem.at[b]
            ).wait()
            pltpu.async_copy(gbuf.at[g], acc_sh.at[colv.at[b]], ssem.at[g], add=True)

        def wait_scatter(jj, b, g):
            pltpu.make_async_copy(
                gbuf.at[g], acc_sh.at[colv.at[b]], ssem.at[g]
            ).wait()

        for b in range(LAG + 2):
            fire_idx(b, b)

        def steady(i, carry):
            for b in range(NSLOT):
                j = i * NSLOT + b

                @pl.when(j >= LAG + 1)
                def _():
                    jj = j - LAG - 1
                    wait_scatter(jj, (b - LAG - 1) % NSLOT, (b - LAG - 1) % GSLOT)

                wait_row(j, b)
                fire_gather(b, b % GSLOT)

                @pl.when(j >= LAG)
                def _():
                    fire_scatter(j - LAG, (b - LAG) % NSLOT, (b - LAG) % GSLOT)

                @pl.when(j + LAG + 2 < nb)
                def _():
                    fire_idx(j + LAG + 2, (b + LAG + 2) % NSLOT)

            return carry

        lax.fori_loop(0, nb // NSLOT, steady, 0)
        for t in range(LAG + 1, 0, -1):
            jj = nb - t
            if t > LAG:
                wait_scatter(jj, jj % NSLOT, jj % GSLOT)
        for t in range(LAG, 0, -1):
            jj = nb - t
            fire_scatter(jj, jj % NSLOT, jj % GSLOT)
        for t in range(LAG, 0, -1):
            jj = nb - t
            wait_scatter(jj, jj % NSLOT, jj % GSLOT)

        plsc.subcore_barrier()
        pltpu.sync_copy(
            acc_sh.at[pl.ds(s * RPT, RPT)],
            out_hbm.at[pl.ds(c * NPAD + s * RPT, RPT)],
        )

    return spmm_kernel(q, row2d, col2d, zrows)


def _scale(deg_blk):
    return lax.rsqrt(1.0 + deg_blk[:, 0:1] + deg_blk[:, 1:2])


def _tc1(x, fc_W, fc_b, W1, degT):
    def body(x_ref, fcw_ref, fcb_ref, w1_ref, deg_ref, q_ref):
        wc = jnp.dot(fcw_ref[...], w1_ref[...], preferred_element_type=jnp.float32)
        bc = jnp.dot(fcb_ref[...], w1_ref[...], preferred_element_type=jnp.float32)
        p1 = jnp.dot(x_ref[...], wc, preferred_element_type=jnp.float32) + bc
        q_ref[...] = p1 * _scale(deg_ref[...])

    return pl.pallas_call(
        body,
        grid=(N // R,),
        in_specs=[
            pl.BlockSpec((R, D), lambda i: (i, 0)),
            pl.BlockSpec((D, D), lambda i: (0, 0)),
            pl.BlockSpec((1, D), lambda i: (0, 0)),
            pl.BlockSpec((D, D), lambda i: (0, 0)),
            pl.BlockSpec((R, NC), lambda i: (i, 0)),
        ],
        out_specs=pl.BlockSpec((R, D), lambda i: (i, 0)),
        out_shape=jax.ShapeDtypeStruct((N, D), jnp.float32),
    )(x, fc_W, fc_b, W1, degT)


def _tc2(agg, q1, degT, b1, W2):
    def body(agg_ref, q1_ref, deg_ref, b1_ref, w2_ref, q2_ref):
        sc = _scale(deg_ref[...])
        a = agg_ref[0] + agg_ref[1] + q1_ref[...]
        h = jnp.maximum(sc * a + b1_ref[...], 0.0)
        p2 = jnp.dot(h, w2_ref[...], preferred_element_type=jnp.float32)
        q2_ref[...] = p2 * sc

    return pl.pallas_call(
        body,
        grid=(N // R,),
        in_specs=[
            pl.BlockSpec((NC, R, D), lambda i: (0, i, 0)),
            pl.BlockSpec((R, D), lambda i: (i, 0)),
            pl.BlockSpec((R, NC), lambda i: (i, 0)),
            pl.BlockSpec((1, D), lambda i: (0, 0)),
            pl.BlockSpec((D, D), lambda i: (0, 0)),
        ],
        out_specs=pl.BlockSpec((R, D), lambda i: (i, 0)),
        out_shape=jax.ShapeDtypeStruct((N, D), jnp.float32),
    )(agg, q1, degT, b1, W2)


def _tc3(agg, q2, degT, b2):
    def body(agg_ref, q2_ref, deg_ref, b2_ref, o_ref):
        sc = _scale(deg_ref[...])
        a = agg_ref[0] + agg_ref[1] + q2_ref[...]
        o_ref[...] = sc * a + b2_ref[...]

    return pl.pallas_call(
        body,
        grid=(N // R,),
        in_specs=[
            pl.BlockSpec((NC, R, D), lambda i: (0, i, 0)),
            pl.BlockSpec((R, D), lambda i: (i, 0)),
            pl.BlockSpec((R, NC), lambda i: (i, 0)),
            pl.BlockSpec((1, D), lambda i: (0, 0)),
        ],
        out_specs=pl.BlockSpec((R, D), lambda i: (i, 0)),
        out_shape=jax.ShapeDtypeStruct((N, D), jnp.float32),
    )(agg, q2, degT, b2)


def kernel(x, edge_index, fc_W, fc_b, W1, b1, W2, b2):
    ei = edge_index.astype(jnp.int32)
    e = ei.shape[1]
    chunk = EB * NW * NSLOT
    epad = (-e) % chunk
    rowf = ei[0]
    colf = ei[1]
    if epad:
        rowf = jnp.concatenate(
            [rowf, jnp.arange(epad, dtype=jnp.int32) % jnp.int32(x.shape[0])]
        )
        colf = jnp.concatenate(
            [colf, N + (jnp.arange(epad, dtype=jnp.int32) % (NPAD - N))]
        )
    er = (e + epad) // EB
    row2d = rowf.reshape(er, EB)
    col2d = colf.reshape(er, EB)

    deg_flat = _sc_degree(col2d)
    degT = deg_flat.reshape(NC, NPAD).T

    zrows = jnp.zeros((RPT, D), jnp.float32)
    q1 = _tc1(x, fc_W, fc_b.reshape(1, D), W1, degT)
    agg1 = _sc_spmm(q1, row2d, col2d, zrows).reshape(NC, NPAD, D)
    q2 = _tc2(agg1, q1, degT, b1.reshape(1, D), W2)
    agg2 = _sc_spmm(q2, row2d, col2d, zrows).reshape(NC, NPAD, D)
    return _tc3(agg2, q2, degT, b2.reshape(1, D))


# speedup vs baseline: 1.0296x; 1.0296x over previous
import functools

import jax
import jax.numpy as jnp
from jax import lax
from jax.experimental import pallas as pl
from jax.experimental.pallas import tpu as pltpu
from jax.experimental.pallas import tpu_sc as plsc

NC = 2
NS = 16
NW = NC * NS
EB = 128
D = 128
N = 10000
RPT = 640
NPAD = NS * RPT
R = 2000
NSLOT = 8
GSLOT = 2
LAG = 1


def _vsmesh():
    return plsc.VectorSubcoreMesh(core_axis_name="c", subcore_axis_name="s")


def _sc_degree(col2d):
    er = col2d.shape[0]
    nb = er // NW

    @functools.partial(
        pl.kernel,
        mesh=_vsmesh(),
        out_type=jax.ShapeDtypeStruct((NC * NPAD,), jnp.float32),
        scratch_types=[
            pltpu.VMEM((NSLOT, EB), jnp.int32),
            pltpu.VMEM((EB,), jnp.float32),
            pltpu.VMEM((RPT,), jnp.float32),
            pltpu.VMEM_SHARED((NPAD,), jnp.float32),
            pltpu.SemaphoreType.DMA((NSLOT,)),
        ],
    )
    def deg_kernel(col_hbm, out_hbm, colv, onesv, stg, deg_sh, csem):
        c = lax.axis_index("c")
        s = lax.axis_index("s")
        wid = s * NC + c
        for j in range(EB // 16):
            onesv[pl.ds(j * 16, 16)] = jnp.ones((16,), jnp.float32)
        for j in range(RPT // 16):
            stg[pl.ds(j * 16, 16)] = jnp.zeros((16,), jnp.float32)
        pltpu.sync_copy(stg, deg_sh.at[pl.ds(s * RPT, RPT)])
        plsc.subcore_barrier()

        def fire_idx(j, b):
            pltpu.async_copy(col_hbm.at[wid + j * NW], colv.at[b], csem.at[b])

        def wait_idx(j, b):
            pltpu.make_async_copy(
                col_hbm.at[wid + j * NW], colv.at[b], csem.at[b]
            ).wait()

        for b in range(NSLOT - 1):
            fire_idx(b, b)

        def body(i, carry):
            for b in range(NSLOT):
                j = i * NSLOT + b
                wait_idx(j, b)
                pltpu.sync_copy(onesv, deg_sh.at[colv.at[b]], add=True)

                @pl.when(j + NSLOT - 1 < nb)
                def _():
                    fire_idx(j + NSLOT - 1, (b + NSLOT - 1) % NSLOT)

            return carry

        lax.fori_loop(0, nb // NSLOT, body, 0)
        plsc.subcore_barrier()
        pltpu.sync_copy(
            deg_sh.at[pl.ds(s * RPT, RPT)],
            out_hbm.at[pl.ds(c * NPAD + s * RPT, RPT)],
        )

    return deg_kernel(col2d)


def _sc_spmm(q, row2d, col2d):
    er = row2d.shape[0]
    nb = er // NW

    @functools.partial(
        pl.kernel,
        mesh=_vsmesh(),
        out_type=jax.ShapeDtypeStruct((NC * NPAD, D), jnp.float32),
        scratch_types=[
            pltpu.VMEM((NSLOT, EB), jnp.int32),
            pltpu.VMEM((NSLOT, EB), jnp.int32),
            pltpu.VMEM((GSLOT, EB, D), jnp.float32),
            pltpu.VMEM_SHARED((NPAD, D), jnp.float32),
            pltpu.SemaphoreType.DMA((NSLOT,)),
            pltpu.SemaphoreType.DMA((NSLOT,)),
            pltpu.SemaphoreType.DMA((GSLOT,)),
            pltpu.SemaphoreType.DMA((GSLOT,)),
        ],
    )
    def spmm_kernel(
        q_hbm, row_hbm, col_hbm, out_hbm, rowv, colv, gbuf, acc_sh,
        rsem, csem, gsem, ssem,
    ):
        c = lax.axis_index("c")
        s = lax.axis_index("s")
        wid = s * NC + c

        def zrow(r, carry):
            for j in range(D // 16):
                gbuf[0, r, pl.ds(j * 16, 16)] = jnp.zeros((16,), jnp.float32)
            return carry

        lax.fori_loop(0, EB, zrow, 0)
        for k in range(RPT // EB):
            pltpu.sync_copy(gbuf.at[0], acc_sh.at[pl.ds(s * RPT + k * EB, EB)])
        plsc.subcore_barrier()

        def fire_idx(j, b):
            pltpu.async_copy(row_hbm.at[wid + j * NW], rowv.at[b], rsem.at[b])
            pltpu.async_copy(col_hbm.at[wid + j * NW], colv.at[b], csem.at[b])

        def wait_row(j, b):
            pltpu.make_async_copy(
                row_hbm.at[wid + j * NW], rowv.at[b], rsem.at[b]
            ).wait()

        def fire_gather(b, g):
            pltpu.async_copy(q_hbm.at[rowv.at[b]], gbuf.at[g], gsem.at[g])

        def fire_scatter(jj, b, g):
            pltpu.make_async_copy(
                q_hbm.at[rowv.at[b]], gbuf.at[g], gsem.at[g]
            ).wait()
            pltpu.make_async_copy(
                col_hbm.at[wid + jj * NW], colv.at[b], csem.at[b]
            ).wait()
            pltpu.async_copy(gbuf.at[g], acc_sh.at[colv.at[b]], ssem.at[g], add=True)

        def wait_scatter(jj, b, g):
            pltpu.make_async_copy(
                gbuf.at[g], acc_sh.at[colv.at[b]], ssem.at[g]
            ).wait()

        for b in range(LAG + 2):
            fire_idx(b, b)

        def steady(i, carry):
            for b in range(NSLOT):
                j = i * NSLOT + b

                @pl.when(j >= LAG + 1)
                def _():
                    jj = j - LAG - 1
                    wait_scatter(jj, (b - LAG - 1) % NSLOT, (b - LAG - 1) % GSLOT)

                wait_row(j, b)
                fire_gather(b, b % GSLOT)

                @pl.when(j >= LAG)
                def _():
                    fire_scatter(j - LAG, (b - LAG) % NSLOT, (b - LAG) % GSLOT)

                @pl.when(j + LAG + 2 < nb)
                def _():
                    fire_idx(j + LAG + 2, (b + LAG + 2) % NSLOT)

            return carry

        lax.fori_loop(0, nb // NSLOT, steady, 0)
        for t in range(LAG + 1, 0, -1):
            jj = nb - t
            if t > LAG:
                wait_scatter(jj, jj % NSLOT, jj % GSLOT)
        for t in range(LAG, 0, -1):
            jj = nb - t
            fire_scatter(jj, jj % NSLOT, jj % GSLOT)
        for t in range(LAG, 0, -1):
            jj = nb - t
            wait_scatter(jj, jj % NSLOT, jj % GSLOT)

        plsc.subcore_barrier()
        pltpu.sync_copy(
            acc_sh.at[pl.ds(s * RPT, RPT)],
            out_hbm.at[pl.ds(c * NPAD + s * RPT, RPT)],
        )

    return spmm_kernel(q, row2d, col2d)


def _scale(deg_blk):
    return lax.rsqrt(1.0 + deg_blk[:, 0:1] + deg_blk[:, 1:2])


def _tc1(x, fc_W, fc_b, W1, degT):
    def body(x_ref, fcw_ref, fcb_ref, w1_ref, deg_ref, q_ref):
        wc = jnp.dot(fcw_ref[...], w1_ref[...], preferred_element_type=jnp.float32)
        bc = jnp.dot(fcb_ref[...], w1_ref[...], preferred_element_type=jnp.float32)
        p1 = jnp.dot(x_ref[...], wc, preferred_element_type=jnp.float32) + bc
        q_ref[...] = p1 * _scale(deg_ref[...])

    return pl.pallas_call(
        body,
        grid=(N // R,),
        in_specs=[
            pl.BlockSpec((R, D), lambda i: (i, 0)),
            pl.BlockSpec((D, D), lambda i: (0, 0)),
            pl.BlockSpec((1, D), lambda i: (0, 0)),
            pl.BlockSpec((D, D), lambda i: (0, 0)),
            pl.BlockSpec((R, NC), lambda i: (i, 0)),
        ],
        out_specs=pl.BlockSpec((R, D), lambda i: (i, 0)),
        out_shape=jax.ShapeDtypeStruct((N, D), jnp.float32),
    )(x, fc_W, fc_b, W1, degT)


def _tc2(agg, q1, degT, b1, W2):
    def body(agg_ref, q1_ref, deg_ref, b1_ref, w2_ref, q2_ref):
        sc = _scale(deg_ref[...])
        a = agg_ref[0] + agg_ref[1] + q1_ref[...]
        h = jnp.maximum(sc * a + b1_ref[...], 0.0)
        p2 = jnp.dot(h, w2_ref[...], preferred_element_type=jnp.float32)
        q2_ref[...] = p2 * sc

    return pl.pallas_call(
        body,
        grid=(N // R,),
        in_specs=[
            pl.BlockSpec((NC, R, D), lambda i: (0, i, 0)),
            pl.BlockSpec((R, D), lambda i: (i, 0)),
            pl.BlockSpec((R, NC), lambda i: (i, 0)),
            pl.BlockSpec((1, D), lambda i: (0, 0)),
            pl.BlockSpec((D, D), lambda i: (0, 0)),
        ],
        out_specs=pl.BlockSpec((R, D), lambda i: (i, 0)),
        out_shape=jax.ShapeDtypeStruct((N, D), jnp.float32),
    )(agg, q1, degT, b1, W2)


def _tc3(agg, q2, degT, b2):
    def body(agg_ref, q2_ref, deg_ref, b2_ref, o_ref):
        sc = _scale(deg_ref[...])
        a = agg_ref[0] + agg_ref[1] + q2_ref[...]
        o_ref[...] = sc * a + b2_ref[...]

    return pl.pallas_call(
        body,
        grid=(N // R,),
        in_specs=[
            pl.BlockSpec((NC, R, D), lambda i: (0, i, 0)),
            pl.BlockSpec((R, D), lambda i: (i, 0)),
            pl.BlockSpec((R, NC), lambda i: (i, 0)),
            pl.BlockSpec((1, D), lambda i: (0, 0)),
        ],
        out_specs=pl.BlockSpec((R, D), lambda i: (i, 0)),
        out_shape=jax.ShapeDtypeStruct((N, D), jnp.float32),
    )(agg, q2, degT, b2)


def kernel(x, edge_index, fc_W, fc_b, W1, b1, W2, b2):
    ei = edge_index.astype(jnp.int32)
    e = ei.shape[1]
    chunk = EB * NW * NSLOT
    epad = (-e) % chunk
    rowf = ei[0]
    colf = ei[1]
    if epad:
        rowf = jnp.concatenate(
            [rowf, jnp.arange(epad, dtype=jnp.int32) % jnp.int32(x.shape[0])]
        )
        colf = jnp.concatenate(
            [colf, N + (jnp.arange(epad, dtype=jnp.int32) % (NPAD - N))]
        )
    er = (e + epad) // EB
    row2d = rowf.reshape(er, EB)
    col2d = colf.reshape(er, EB)

    deg_flat = _sc_degree(col2d)
    degT = deg_flat.reshape(NC, NPAD).T

    q1 = _tc1(x, fc_W, fc_b.reshape(1, D), W1, degT)
    agg1 = _sc_spmm(q1, row2d, col2d).reshape(NC, NPAD, D)
    q2 = _tc2(agg1, q1, degT, b1.reshape(1, D), W2)
    agg2 = _sc_spmm(q2, row2d, col2d).reshape(NC, NPAD, D)
    return _tc3(agg2, q2, degT, b2.reshape(1, D))
